# baseline (device time: 358988 ns/iter reference)
import jax
import jax.numpy as jnp
from jax import lax
from jax.experimental import pallas as pl
from jax.experimental.pallas import tpu as pltpu

N_DEV = 16
N_SUB = 4
N_RINGS = 2 * N_SUB
N_SLOTS = 3


def kernel(x, w_mat):
    m, k_per = x.shape
    _, n = w_mat.shape
    m_per = m // N_DEV
    n_half = n // 2
    n_sub = n_half // N_SUB

    def body(x_ref, w_ref, out_ref, *scratch):
        comms = scratch[0:N_RINGS]
        ssems = scratch[N_RINGS:2 * N_RINGS]
        rsems = scratch[2 * N_RINGS:3 * N_RINGS]
        credits = scratch[3 * N_RINGS:4 * N_RINGS]

        my = lax.axis_index("i")
        left = lax.rem(my + N_DEV - 1, N_DEV)
        right = lax.rem(my + 1, N_DEV)

        rings = []
        for k in range(N_SUB):
            rings.append((comms[k], ssems[k], rsems[k], credits[k],
                          right, left, k * n_sub))
        for k in range(N_SUB):
            j = N_SUB + k
            rings.append((comms[j], ssems[j], rsems[j], credits[j],
                          left, right, n_half + k * n_sub))

        barrier_sem = pltpu.get_barrier_semaphore()
        for nbr in (left, right):
            pl.semaphore_signal(barrier_sem, inc=1, device_id=(nbr,),
                                device_id_type=pl.DeviceIdType.MESH)
        pl.semaphore_wait(barrier_sem, 2)

        def partial_a(c):
            xa = x_ref[pl.ds(c * m_per, m_per), :]
            return jnp.dot(xa, w_ref[:, 0:n_half],
                           preferred_element_type=jnp.float32)

        def partial_b(c):
            xa = x_ref[pl.ds(c * m_per, m_per), :]
            return jnp.dot(xa, w_ref[:, n_half:n],
                           preferred_element_type=jnp.float32)

        def split(pa, pb):
            return tuple(pa[:, k * n_sub:(k + 1) * n_sub] for k in range(N_SUB)) + \
                   tuple(pb[:, k * n_sub:(k + 1) * n_sub] for k in range(N_SUB))

        pa = partial_a(lax.rem(my + N_DEV - 1, N_DEV)).astype(jnp.bfloat16)
        pb = partial_b(lax.rem(my + 1, N_DEV)).astype(jnp.bfloat16)
        for i, blk in enumerate(split(pa, pb)):
            rings[i][0][0, :, :] = blk

        inflight = [[] for _ in rings]
        for i, (comm, ssem, rsem, _, down, _, _) in enumerate(rings):
            rdma = pltpu.make_async_remote_copy(
                src_ref=comm.at[0], dst_ref=comm.at[1],
                send_sem=ssem.at[0], recv_sem=rsem.at[1],
                device_id=(down,), device_id_type=pl.DeviceIdType.MESH,
            )
            rdma.start()
            inflight[i].append(rdma)

        for h in range(N_DEV - 1):
            r = (h + 1) % N_SLOTS
            d2 = (h + 2) % N_SLOTS
            part_a = partial_a(lax.rem(my + 2 * N_DEV - 2 - h, N_DEV))
            part_b = partial_b(lax.rem(my + 2 + h, N_DEV))
            parts = split(part_a, part_b)
            for i, (comm, ssem, rsem, credit, down, up, col) in enumerate(rings):
                if h >= 1:
                    inflight[i].pop(0).wait_send()
                    if h <= N_DEV - 3:
                        pl.semaphore_signal(
                            credit, inc=1, device_id=(up,),
                            device_id_type=pl.DeviceIdType.MESH)
                recv = pltpu.make_async_remote_copy(
                    src_ref=comm.at[d2], dst_ref=comm.at[r],
                    send_sem=ssem.at[d2], recv_sem=rsem.at[r],
                    device_id=(up,), device_id_type=pl.DeviceIdType.MESH,
                )
                recv.wait_recv()
                acc = comm[r, :, :].astype(jnp.float32) + parts[i].astype(jnp.float32)
                if h < N_DEV - 2:
                    comm[r, :, :] = acc.astype(jnp.bfloat16)
                    if h >= 1:
                        pl.semaphore_wait(credit, 1)
                    rdma = pltpu.make_async_remote_copy(
                        src_ref=comm.at[r], dst_ref=comm.at[d2],
                        send_sem=ssem.at[r], recv_sem=rsem.at[d2],
                        device_id=(down,), device_id_type=pl.DeviceIdType.MESH,
                    )
                    rdma.start()
                    inflight[i].append(rdma)
                else:
                    y = acc
                    c = 0.7978845608028654
                    out_ref[:, pl.ds(col, n_sub)] = (
                        0.5 * y * (1.0 + jnp.tanh(c * (y + 0.044715 * y * y * y))))

        for q in inflight:
            for rdma in q:
                rdma.wait_send()

    scratch_shapes = (
        [pltpu.VMEM((N_SLOTS, m_per, n_sub), jnp.bfloat16)] * N_RINGS
        + [pltpu.SemaphoreType.DMA((N_SLOTS,))] * N_RINGS
        + [pltpu.SemaphoreType.DMA((N_SLOTS,))] * N_RINGS
        + [pltpu.SemaphoreType.REGULAR] * N_RINGS
    )
    return pl.pallas_call(
        body,
        out_shape=jax.ShapeDtypeStruct((m_per, n), jnp.float32),
        in_specs=[
            pl.BlockSpec(memory_space=pltpu.VMEM),
            pl.BlockSpec(memory_space=pltpu.VMEM),
        ],
        out_specs=pl.BlockSpec(memory_space=pltpu.VMEM),
        scratch_shapes=scratch_shapes,
        compiler_params=pltpu.CompilerParams(collective_id=0),
    )(x, w_mat)


# device time: 358862 ns/iter; 1.0004x vs baseline; 1.0004x over previous
import jax
import jax.numpy as jnp
from jax import lax
from jax.experimental import pallas as pl
from jax.experimental.pallas import tpu as pltpu

N_DEV = 16
N_SUB = 2
N_RINGS = 2 * N_SUB
N_SLOTS = 3


def kernel(x, w_mat):
    m, k_per = x.shape
    _, n = w_mat.shape
    m_per = m // N_DEV
    n_half = n // 2
    n_sub = n_half // N_SUB

    def body(x_ref, w_ref, out_ref, *scratch):
        comms = scratch[0:N_RINGS]
        ssems = scratch[N_RINGS:2 * N_RINGS]
        rsems = scratch[2 * N_RINGS:3 * N_RINGS]
        credits = scratch[3 * N_RINGS:4 * N_RINGS]

        my = lax.axis_index("i")
        left = lax.rem(my + N_DEV - 1, N_DEV)
        right = lax.rem(my + 1, N_DEV)

        rings = []
        for k in range(N_SUB):
            rings.append((comms[k], ssems[k], rsems[k], credits[k],
                          right, left, k * n_sub))
        for k in range(N_SUB):
            j = N_SUB + k
            rings.append((comms[j], ssems[j], rsems[j], credits[j],
                          left, right, n_half + k * n_sub))

        barrier_sem = pltpu.get_barrier_semaphore()
        for nbr in (left, right):
            pl.semaphore_signal(barrier_sem, inc=1, device_id=(nbr,),
                                device_id_type=pl.DeviceIdType.MESH)
        pl.semaphore_wait(barrier_sem, 2)

        def partial_a(c):
            xa = x_ref[pl.ds(c * m_per, m_per), :]
            return jnp.dot(xa, w_ref[:, 0:n_half],
                           preferred_element_type=jnp.float32)

        def partial_b(c):
            xa = x_ref[pl.ds(c * m_per, m_per), :]
            return jnp.dot(xa, w_ref[:, n_half:n],
                           preferred_element_type=jnp.float32)

        def split(pa, pb):
            return tuple(pa[:, k * n_sub:(k + 1) * n_sub] for k in range(N_SUB)) + \
                   tuple(pb[:, k * n_sub:(k + 1) * n_sub] for k in range(N_SUB))

        pa = partial_a(lax.rem(my + N_DEV - 1, N_DEV)).astype(jnp.bfloat16)
        pb = partial_b(lax.rem(my + 1, N_DEV)).astype(jnp.bfloat16)
        for i, blk in enumerate(split(pa, pb)):
            rings[i][0][0, :, :] = blk

        inflight = [[] for _ in rings]
        for i, (comm, ssem, rsem, _, down, _, _) in enumerate(rings):
            rdma = pltpu.make_async_remote_copy(
                src_ref=comm.at[0], dst_ref=comm.at[1],
                send_sem=ssem.at[0], recv_sem=rsem.at[1],
                device_id=(down,), device_id_type=pl.DeviceIdType.MESH,
            )
            rdma.start()
            inflight[i].append(rdma)

        for h in range(N_DEV - 1):
            r = (h + 1) % N_SLOTS
            d2 = (h + 2) % N_SLOTS
            part_a = partial_a(lax.rem(my + 2 * N_DEV - 2 - h, N_DEV))
            part_b = partial_b(lax.rem(my + 2 + h, N_DEV))
            parts = split(part_a, part_b)
            for i, (comm, ssem, rsem, credit, down, up, col) in enumerate(rings):
                if h >= 1:
                    inflight[i].pop(0).wait_send()
                    if h <= N_DEV - 3:
                        pl.semaphore_signal(
                            credit, inc=1, device_id=(up,),
                            device_id_type=pl.DeviceIdType.MESH)
                recv = pltpu.make_async_remote_copy(
                    src_ref=comm.at[d2], dst_ref=comm.at[r],
                    send_sem=ssem.at[d2], recv_sem=rsem.at[r],
                    device_id=(up,), device_id_type=pl.DeviceIdType.MESH,
                )
                recv.wait_recv()
                acc = comm[r, :, :].astype(jnp.float32) + parts[i].astype(jnp.float32)
                if h < N_DEV - 2:
                    comm[r, :, :] = acc.astype(jnp.bfloat16)
                    if h >= 1:
                        pl.semaphore_wait(credit, 1)
                    rdma = pltpu.make_async_remote_copy(
                        src_ref=comm.at[r], dst_ref=comm.at[d2],
                        send_sem=ssem.at[r], recv_sem=rsem.at[d2],
                        device_id=(down,), device_id_type=pl.DeviceIdType.MESH,
                    )
                    rdma.start()
                    inflight[i].append(rdma)
                else:
                    y = acc
                    c = 0.7978845608028654
                    out_ref[:, pl.ds(col, n_sub)] = (
                        0.5 * y * (1.0 + jnp.tanh(c * (y + 0.044715 * y * y * y))))

        for q in inflight:
            for rdma in q:
                rdma.wait_send()

    scratch_shapes = (
        [pltpu.VMEM((N_SLOTS, m_per, n_sub), jnp.bfloat16)] * N_RINGS
        + [pltpu.SemaphoreType.DMA((N_SLOTS,))] * N_RINGS
        + [pltpu.SemaphoreType.DMA((N_SLOTS,))] * N_RINGS
        + [pltpu.SemaphoreType.REGULAR] * N_RINGS
    )
    return pl.pallas_call(
        body,
        out_shape=jax.ShapeDtypeStruct((m_per, n), jnp.float32),
        in_specs=[
            pl.BlockSpec(memory_space=pltpu.VMEM),
            pl.BlockSpec(memory_space=pltpu.VMEM),
        ],
        out_specs=pl.BlockSpec(memory_space=pltpu.VMEM),
        scratch_shapes=scratch_shapes,
        compiler_params=pltpu.CompilerParams(collective_id=0),
    )(x, w_mat)
